# Initial kernel scaffold; baseline (speedup 1.0000x reference)
import jax
import jax.numpy as jnp
from jax.experimental import pallas as pl


def kernel(x, Wg, bg, We, be, context_length, horizon, top_k):
    raise NotImplementedError("write your pallas kernel here")


# speedup vs baseline: 2.4416x; 2.4416x over previous
import functools

import jax
import jax.numpy as jnp
from jax.experimental import pallas as pl
from jax.experimental.pallas import tpu as pltpu


def _router_body(x_ref, wg_ref, bg_ref, wall_ref, be_ref, out_ref, *, E, H):
    xb = x_ref[...]
    logits = jnp.dot(xb, wg_ref[...], preferred_element_type=jnp.float32)
    logits = logits + bg_ref[...]
    m = jnp.max(logits, axis=-1, keepdims=True)
    ex = jnp.exp(logits - m)
    probs = ex / jnp.sum(ex, axis=-1, keepdims=True)

    eidx = jax.lax.broadcasted_iota(jnp.int32, probs.shape, 1)
    m1 = jnp.max(probs, axis=-1, keepdims=True)
    idx1 = jnp.min(jnp.where(probs == m1, eidx, E), axis=-1, keepdims=True)
    probs_m = jnp.where(eidx == idx1, -jnp.inf, probs)
    m2 = jnp.max(probs_m, axis=-1, keepdims=True)
    idx2 = jnp.min(jnp.where(probs_m == m2, eidx, E), axis=-1, keepdims=True)

    s = m1 + m2
    inv = 1.0 / (s + 1e-8)
    w1 = jnp.where(s <= 0, 0.5, m1 * inv)
    w2 = jnp.where(s <= 0, 0.5, m2 * inv)
    c = w1 * (eidx == idx1).astype(jnp.float32) + w2 * (eidx == idx2).astype(jnp.float32)

    preds = jnp.dot(xb, wall_ref[...], preferred_element_type=jnp.float32)

    acc = jnp.dot(c, be_ref[...], preferred_element_type=jnp.float32)
    for e in range(E):
        acc = acc + c[:, e:e + 1] * preds[:, e * H:(e + 1) * H]
    out_ref[...] = acc


def kernel(x, Wg, bg, We, be, context_length, horizon, top_k):
    B, C = x.shape
    E, _, H = We.shape
    W_all = jnp.transpose(We, (1, 0, 2)).reshape(C, E * H)
    bg2 = bg.reshape(1, E)

    Bb = 512
    grid = (B // Bb,)
    body = functools.partial(_router_body, E=E, H=H)
    return pl.pallas_call(
        body,
        grid=grid,
        in_specs=[
            pl.BlockSpec((Bb, C), lambda i: (i, 0)),
            pl.BlockSpec((C, E), lambda i: (0, 0)),
            pl.BlockSpec((1, E), lambda i: (0, 0)),
            pl.BlockSpec((C, E * H), lambda i: (0, 0)),
            pl.BlockSpec((E, H), lambda i: (0, 0)),
        ],
        out_specs=pl.BlockSpec((Bb, H), lambda i: (i, 0)),
        out_shape=jax.ShapeDtypeStruct((B, H), jnp.float32),
    )(x, Wg, bg2, W_all, be)


# speedup vs baseline: 2.4919x; 1.0206x over previous
import functools

import jax
import jax.numpy as jnp
from jax.experimental import pallas as pl
from jax.experimental.pallas import tpu as pltpu


def _router_body(x_ref, wg_ref, bg_ref, wall_ref, be_ref, out_ref, *, E, H):
    xb = x_ref[...]
    logits = jnp.dot(xb, wg_ref[...], preferred_element_type=jnp.float32)
    logits = logits + bg_ref[...]
    m = jnp.max(logits, axis=-1, keepdims=True)
    ex = jnp.exp(logits - m)
    probs = ex / jnp.sum(ex, axis=-1, keepdims=True)

    eidx = jax.lax.broadcasted_iota(jnp.int32, probs.shape, 1)
    m1 = jnp.max(probs, axis=-1, keepdims=True)
    idx1 = jnp.min(jnp.where(probs == m1, eidx, E), axis=-1, keepdims=True)
    probs_m = jnp.where(eidx == idx1, -jnp.inf, probs)
    m2 = jnp.max(probs_m, axis=-1, keepdims=True)
    idx2 = jnp.min(jnp.where(probs_m == m2, eidx, E), axis=-1, keepdims=True)

    s = m1 + m2
    inv = 1.0 / (s + 1e-8)
    w1 = jnp.where(s <= 0, 0.5, m1 * inv)
    w2 = jnp.where(s <= 0, 0.5, m2 * inv)
    c = w1 * (eidx == idx1).astype(jnp.float32) + w2 * (eidx == idx2).astype(jnp.float32)

    xb_bf = xb.astype(jnp.bfloat16)
    preds = jnp.dot(xb_bf, wall_ref[...], preferred_element_type=jnp.float32)

    acc = jnp.dot(c, be_ref[...], preferred_element_type=jnp.float32)
    for e in range(E):
        acc = acc + c[:, e:e + 1] * preds[:, e * H:(e + 1) * H]
    out_ref[...] = acc


def kernel(x, Wg, bg, We, be, context_length, horizon, top_k):
    B, C = x.shape
    E, _, H = We.shape
    W_all = jnp.transpose(We, (1, 0, 2)).reshape(C, E * H).astype(jnp.bfloat16)
    bg2 = bg.reshape(1, E)

    Bb = 512
    grid = (B // Bb,)
    body = functools.partial(_router_body, E=E, H=H)
    return pl.pallas_call(
        body,
        grid=grid,
        in_specs=[
            pl.BlockSpec((Bb, C), lambda i: (i, 0)),
            pl.BlockSpec((C, E), lambda i: (0, 0)),
            pl.BlockSpec((1, E), lambda i: (0, 0)),
            pl.BlockSpec((C, E * H), lambda i: (0, 0)),
            pl.BlockSpec((E, H), lambda i: (0, 0)),
        ],
        out_specs=pl.BlockSpec((Bb, H), lambda i: (i, 0)),
        out_shape=jax.ShapeDtypeStruct((B, H), jnp.float32),
    )(x, Wg, bg2, W_all, be)


# speedup vs baseline: 2.4935x; 1.0006x over previous
import functools

import jax
import jax.numpy as jnp
from jax.experimental import pallas as pl
from jax.experimental.pallas import tpu as pltpu


def _router_body(x0_ref, x1_ref, wg_ref, bg_ref, wall_ref, be_ref, out_ref, *, E, H):
    xb = jnp.concatenate([x0_ref[...], x1_ref[...]], axis=0)
    logits = jnp.dot(xb, wg_ref[...], preferred_element_type=jnp.float32)
    logits = logits + bg_ref[...]
    m = jnp.max(logits, axis=-1, keepdims=True)
    ex = jnp.exp(logits - m)
    probs = ex / jnp.sum(ex, axis=-1, keepdims=True)

    eidx = jax.lax.broadcasted_iota(jnp.int32, probs.shape, 1)
    m1 = jnp.max(probs, axis=-1, keepdims=True)
    idx1 = jnp.min(jnp.where(probs == m1, eidx, E), axis=-1, keepdims=True)
    probs_m = jnp.where(eidx == idx1, -jnp.inf, probs)
    m2 = jnp.max(probs_m, axis=-1, keepdims=True)
    idx2 = jnp.min(jnp.where(probs_m == m2, eidx, E), axis=-1, keepdims=True)

    s = m1 + m2
    inv = 1.0 / (s + 1e-8)
    w1 = jnp.where(s <= 0, 0.5, m1 * inv)
    w2 = jnp.where(s <= 0, 0.5, m2 * inv)
    c = w1 * (eidx == idx1).astype(jnp.float32) + w2 * (eidx == idx2).astype(jnp.float32)

    xb_bf = xb.astype(jnp.bfloat16)
    preds = jnp.dot(xb_bf, wall_ref[...], preferred_element_type=jnp.float32)

    acc = jnp.dot(c, be_ref[...], preferred_element_type=jnp.float32)
    for e in range(E):
        acc = acc + c[:, e:e + 1] * preds[:, e * H:(e + 1) * H]
    out_ref[...] = acc


def kernel(x, Wg, bg, We, be, context_length, horizon, top_k):
    B, C = x.shape
    E, _, H = We.shape
    W_all = jnp.transpose(We, (1, 0, 2)).reshape(C, E * H).astype(jnp.bfloat16)
    bg2 = bg.reshape(1, E)

    Bb = 512
    grid = (B // Bb,)
    body = functools.partial(_router_body, E=E, H=H)
    return pl.pallas_call(
        body,
        grid=grid,
        in_specs=[
            pl.BlockSpec((Bb // 2, C), lambda i: (2 * i, 0)),
            pl.BlockSpec((Bb // 2, C), lambda i: (2 * i + 1, 0)),
            pl.BlockSpec((C, E), lambda i: (0, 0)),
            pl.BlockSpec((1, E), lambda i: (0, 0)),
            pl.BlockSpec((C, E * H), lambda i: (0, 0)),
            pl.BlockSpec((E, H), lambda i: (0, 0)),
        ],
        out_specs=pl.BlockSpec((Bb, H), lambda i: (i, 0)),
        out_shape=jax.ShapeDtypeStruct((B, H), jnp.float32),
    )(x, x, Wg, bg2, W_all, be)
